# baseline (speedup 1.0000x reference)
import functools

import jax
import jax.numpy as jnp
from jax import lax
from jax.experimental import pallas as pl
from jax.experimental.pallas import tpu as pltpu
from jax.experimental.pallas import tpu_sc as plsc

N = 10000
E = 320000
D = 128
G = 16
K = 30
RATIO = 0.5

NP = 10240
CH = 128
NW = 32
NCHUNK = 80
EW = NCHUNK * CH
EP = NW * EW
ROWS_W = NP // 16

BI = 512
NBLK = NP // BI
BR = 512
NRBLK = NP // BR
PH = 20

_mesh = plsc.VectorSubcoreMesh(core_axis_name="c", subcore_axis_name="s")


def _zfill16(ref, nwords, value=0.0):
    v = jnp.full((16,), value, jnp.float32)

    def body(i, c):
        ref[pl.ds(i * 16, 16)] = v
        return c

    lax.fori_loop(0, nwords // 16, body, 0)


def _zfill2d(ref):
    z = jnp.zeros((16,), jnp.float32)

    def body(i, c):
        r = i // 8
        q = i % 8
        ref[r, pl.ds(q * 16, 16)] = z
        return c

    lax.fori_loop(0, 128 * 8, body, 0)


@functools.partial(
    pl.kernel,
    out_type=jax.ShapeDtypeStruct((2, NP), jnp.float32),
    mesh=_mesh,
    scratch_types=[
        pltpu.VMEM((NCHUNK, CH), jnp.int32),
        pltpu.VMEM((CH,), jnp.float32),
        pltpu.VMEM((ROWS_W,), jnp.float32),
        pltpu.VMEM_SHARED((NP,), jnp.float32),
        pltpu.SemaphoreType.DMA,
    ],
)
def _sc_deg(dst_hbm, out_hbm, didx, ones, zbuf, acc, sem):
    cid = lax.axis_index("c")
    sid = lax.axis_index("s")
    wid = sid * 2 + cid
    _zfill16(ones, CH, 1.0)
    _zfill16(zbuf, ROWS_W, 0.0)
    pltpu.sync_copy(zbuf, acc.at[pl.ds(sid * ROWS_W, ROWS_W)])
    plsc.subcore_barrier()
    pltpu.sync_copy(dst_hbm.at[pl.ds(wid * NCHUNK, NCHUNK)], didx)

    pend = []
    for ph in range(NCHUNK // PH):
        fired = [pltpu.async_copy(ones, acc.at[didx.at[ph * PH + k]], sem, add=True)
                 for k in range(PH)]
        for cp in pend:
            cp.wait()
        pend = fired
    for cp in pend:
        cp.wait()
    plsc.subcore_barrier()
    pltpu.sync_copy(acc.at[pl.ds(sid * ROWS_W, ROWS_W)],
                    out_hbm.at[cid, pl.ds(sid * ROWS_W, ROWS_W)])


@functools.partial(
    pl.kernel,
    out_type=jax.ShapeDtypeStruct((2, NP), jnp.float32),
    mesh=_mesh,
    scratch_types=[
        pltpu.VMEM((NCHUNK, CH), jnp.int32),
        pltpu.VMEM((NCHUNK, CH), jnp.int32),
        pltpu.VMEM((NCHUNK, CH), jnp.float32),
        pltpu.VMEM((ROWS_W,), jnp.float32),
        pltpu.VMEM_SHARED((NP,), jnp.float32),
        pltpu.SemaphoreType.DMA,
        pltpu.SemaphoreType.DMA,
    ],
)
def _sc_scal(t_hbm, src_hbm, dst_hbm, out_hbm, sidx, didx, vals, zbuf, acc, semg, sems):
    cid = lax.axis_index("c")
    sid = lax.axis_index("s")
    wid = sid * 2 + cid
    _zfill16(zbuf, ROWS_W, 0.0)
    pltpu.sync_copy(zbuf, acc.at[pl.ds(sid * ROWS_W, ROWS_W)])
    plsc.subcore_barrier()
    pltpu.sync_copy(src_hbm.at[pl.ds(wid * NCHUNK, NCHUNK)], sidx)
    pltpu.sync_copy(dst_hbm.at[pl.ds(wid * NCHUNK, NCHUNK)], didx)

    cp_g = {}
    cp_s = {}
    for j in range(PH):
        cp_g[j] = pltpu.async_copy(t_hbm.at[sidx.at[j]], vals.at[j], semg)
    for j in range(NCHUNK):
        cp_g[j].wait()
        cp_s[j] = pltpu.async_copy(vals.at[j], acc.at[didx.at[j]], sems, add=True)
        if j + PH < NCHUNK:
            cp_g[j + PH] = pltpu.async_copy(t_hbm.at[sidx.at[j + PH]], vals.at[j + PH], semg)
    for j in range(NCHUNK):
        cp_s[j].wait()
    plsc.subcore_barrier()
    pltpu.sync_copy(acc.at[pl.ds(sid * ROWS_W, ROWS_W)],
                    out_hbm.at[cid, pl.ds(sid * ROWS_W, ROWS_W)])


NHALF = NCHUNK // 2


@functools.partial(
    pl.kernel,
    out_type=jax.ShapeDtypeStruct((2, NP, D), jnp.float32),
    mesh=_mesh,
    scratch_types=[
        pltpu.VMEM((NHALF, CH), jnp.int32),
        pltpu.VMEM((NHALF, CH), jnp.int32),
        pltpu.VMEM((CH, D), jnp.float32),
        pltpu.VMEM((CH, D), jnp.float32),
        pltpu.VMEM_SHARED((NP, D), jnp.float32),
        pltpu.SemaphoreType.DMA,
        pltpu.SemaphoreType.DMA,
        pltpu.SemaphoreType.DMA,
        pltpu.SemaphoreType.DMA,
    ],
)
def _sc_dense(hp_hbm, src_hbm, dst_hbm, out_hbm, sidx, didx, rows0, rows1,
              acc, semg0, semg1, sems0, sems1):
    cid = lax.axis_index("c")
    sid = lax.axis_index("s")
    wid = sid * 2 + cid
    _zfill2d(rows0)
    for kk in range(ROWS_W // CH):
        pltpu.sync_copy(rows0, acc.at[pl.ds(sid * ROWS_W + kk * CH, CH)])
    plsc.subcore_barrier()

    bufs = (rows0, rows1)
    semg = (semg0, semg1)
    sems = (sems0, sems1)
    for half in range(2):
        base = wid * NCHUNK + half * NHALF
        pltpu.sync_copy(src_hbm.at[pl.ds(base, NHALF)], sidx)
        pltpu.sync_copy(dst_hbm.at[pl.ds(base, NHALF)], didx)
        cp_g = {}
        cp_s = {}
        cp_g[0] = pltpu.async_copy(hp_hbm.at[sidx.at[0]], bufs[0], semg[0])
        for j in range(NHALF):
            p = j % 2
            if j >= 1:
                cp_s[j - 1].wait()
            if j + 1 < NHALF:
                cp_g[j + 1] = pltpu.async_copy(hp_hbm.at[sidx.at[j + 1]], bufs[1 - p], semg[1 - p])
            cp_g[j].wait()
            cp_s[j] = pltpu.async_copy(bufs[p], acc.at[didx.at[j]], sems[p], add=True)
        cp_s[NHALF - 1].wait()
    plsc.subcore_barrier()
    pltpu.sync_copy(acc.at[pl.ds(sid * ROWS_W, ROWS_W)],
                    out_hbm.at[cid, pl.ds(sid * ROWS_W, ROWS_W)])


def _elu(a):
    return jnp.where(a > 0, a, jnp.exp(a) - 1.0)


def _to_col(row):
    B = row.shape[1]
    eye = (lax.broadcasted_iota(jnp.int32, (B, B), 0)
           == lax.broadcasted_iota(jnp.int32, (B, B), 1))
    z = jnp.zeros((B, B), row.dtype)
    return jnp.sum(jnp.where(eye, jnp.broadcast_to(row, (B, B)), z),
                   axis=1, keepdims=True)


def _to_row(col):
    B = col.shape[0]
    eye = (lax.broadcasted_iota(jnp.int32, (B, B), 0)
           == lax.broadcasted_iota(jnp.int32, (B, B), 1))
    z = jnp.zeros((B, B), col.dtype)
    return jnp.sum(jnp.where(eye, jnp.broadcast_to(col, (B, B)), z),
                   axis=0, keepdims=True)


def _starts_body(bT_ref, st_ref):
    def body(t, acc):
        blk = bT_ref[:, pl.ds(t * BI, BI)]
        gi = lax.broadcasted_iota(jnp.int32, (32, BI), 0)
        lt = (jnp.broadcast_to(blk, (32, BI)) < gi).astype(jnp.int32)
        return acc + jnp.sum(lt, axis=1, keepdims=True)

    st = lax.fori_loop(0, NBLK, body, jnp.zeros((32, 1), jnp.int32))
    st_ref[...] = _to_row(st)


def _m1_body(x_ref, w_ref, degp_ref, hp_ref, inv_ref):
    deg = degp_ref[0:1, :] + degp_ref[1:2, :] + 1.0
    inv_row = lax.rsqrt(deg)
    inv_ref[...] = inv_row
    h = jnp.dot(x_ref[...], w_ref[...], preferred_element_type=jnp.float32)
    hp_ref[...] = h * _to_col(inv_row)


def _conv_fin_body(acc_ref, hp_ref, inv_ref, b_ref, wp_ref, x_ref, tp_ref):
    inv_col = _to_col(inv_ref[...])
    a = inv_col * (acc_ref[0] + acc_ref[1] + hp_ref[...]) + b_ref[...]
    x = _elu(a)
    x_ref[...] = x
    t = jnp.sum(x * wp_ref[...], axis=1, keepdims=True) * inv_col
    tp_ref[...] = _to_row(t)


def _sasm_body(sacc_ref, tp_ref, inv_ref, b_ref, s_ref):
    s_ref[...] = (inv_ref[...] * (sacc_ref[0:1, :] + sacc_ref[1:2, :] + tp_ref[...])
                  + b_ref[...])


def _rank_body(s_ref, b_ref, sT_ref, bT_ref, st_ref, rank_ref):
    i = pl.program_id(0)
    si = _to_col(s_ref[...])
    bi = _to_col(b_ref[...])
    glo = jnp.min(b_ref[...])
    ghi = jnp.max(b_ref[...])
    gidx = lax.broadcasted_iota(jnp.int32, (1, 32), 1)
    srow = st_ref[...]
    z = jnp.zeros((1, 32), jnp.int32)
    jlo_val = jnp.sum(jnp.where(gidx == glo, srow, z))
    jhi_val = jnp.sum(jnp.where(gidx == ghi + 1, srow, z))
    jlo = jlo_val // BR
    njb = (jhi_val + BR - 1) // BR - jlo
    ii = lax.broadcasted_iota(jnp.int32, (BR, 1), 0) + i * BR

    def body(kk, cnt):
        j0 = pl.multiple_of((jlo + kk) * BR, BR)
        sj = sT_ref[:, pl.ds(j0, BR)]
        bj = bT_ref[:, pl.ds(j0, BR)]
        jj = lax.broadcasted_iota(jnp.int32, (1, BR), 1) + j0
        ahead = (sj > si) | ((sj == si) & (jj < ii))
        m = (bj == bi) & ahead
        return cnt + jnp.sum(m.astype(jnp.float32), axis=1, keepdims=True)

    cnt = lax.fori_loop(0, njb, body, jnp.zeros((BR, 1), jnp.float32))
    rank_ref[...] = _to_row(cnt.astype(jnp.int32))


def _gate(s_row, rank_row, b_row, tg_ref):
    geq = (lax.broadcasted_iota(jnp.int32, (G, BI), 0)
           == jnp.broadcast_to(b_row, (G, BI)))
    zf = jnp.zeros((G, BI), jnp.float32)
    thr = jnp.sum(jnp.where(geq, jnp.broadcast_to(tg_ref[...], (G, BI)), zf),
                  axis=0, keepdims=True)
    keep = (rank_row.astype(jnp.float32) < thr).astype(jnp.float32)
    return _to_col(jnp.tanh(s_row) * keep)


def _gate_mm_body(x_ref, s_ref, rank_ref, b_ref, tg_ref, inv_ref, w_ref, out_ref):
    gcol = _gate(s_ref[...], rank_ref[...], b_ref[...], tg_ref)
    xp = x_ref[...] * gcol
    out_ref[...] = (jnp.dot(xp, w_ref[...], preferred_element_type=jnp.float32)
                    * _to_col(inv_ref[...]))


def _gate_mv_body(x_ref, s_ref, rank_ref, b_ref, tg_ref, inv_ref, w_ref, tp_ref):
    gcol = _gate(s_ref[...], rank_ref[...], b_ref[...], tg_ref)
    xp = x_ref[...] * gcol
    t = jnp.sum(xp * w_ref[...], axis=1, keepdims=True) * _to_col(inv_ref[...])
    tp_ref[...] = _to_row(t)


def _final_body(x3T_ref, bT_ref, r3_ref, wc1_ref, bc1_ref, wc2_ref, bc2_ref, out_ref):
    def body(jb, acc):
        j0 = pl.multiple_of(jb * BI, BI)
        xr = x3T_ref[:, pl.ds(j0, BI)]
        bt = bT_ref[:, pl.ds(j0, BI)]
        rk = _to_col(r3_ref[:, pl.ds(j0, BI)])
        gi = lax.broadcasted_iota(jnp.int32, (G, BI), 0)
        A = jnp.where(bt == gi, jnp.broadcast_to(xr, (G, BI)), 0.0)
        ki = lax.broadcasted_iota(jnp.int32, (BI, K), 1)
        B = (rk == ki).astype(jnp.float32)
        return acc + jnp.dot(A, B, preferred_element_type=jnp.float32)

    pooled = lax.fori_loop(0, NBLK, body, jnp.zeros((G, K), jnp.float32))
    h = jnp.dot(pooled, wc1_ref[...], preferred_element_type=jnp.float32) + bc1_ref[...]
    h = _elu(h)
    out_ref[...] = jnp.dot(h, wc2_ref[...], preferred_element_type=jnp.float32) + bc2_ref[...]


def _row(i):
    return pl.BlockSpec((1, BI), lambda i: (0, i))


def _feat(i):
    return pl.BlockSpec((BI, D), lambda i: (i, 0))


def _full(shape):
    return pl.BlockSpec(shape, lambda i: tuple(0 for _ in shape))


def kernel(x, edge_index, batch, W1, b1, Wp1, bp1, W2, b2, Wp2, bp2, W3, b3, Wc1, bc1, Wc2, bc2):
    f32 = jnp.float32
    i32 = jnp.int32
    src = edge_index[0].astype(i32)
    dst = edge_index[1].astype(i32)
    pe = EP - E
    pad_ar = jnp.arange(pe, dtype=i32)
    src_p = jnp.concatenate([src, pad_ar % N]).reshape(EP // CH, CH)
    dst_p = jnp.concatenate([dst, N + pad_ar % (NP - N)]).reshape(EP // CH, CH)

    xp = jnp.pad(x, ((0, NP - N), (0, 0)))
    batch_p = jnp.concatenate([batch.astype(i32), jnp.full((NP - N,), G, i32)])
    bT = batch_p.reshape(1, NP)

    grid = (NBLK,)

    starts = pl.pallas_call(
        _starts_body,
        in_specs=[pl.BlockSpec((1, NP), lambda: (0, 0))],
        out_specs=pl.BlockSpec((1, 32), lambda: (0, 0)),
        out_shape=jax.ShapeDtypeStruct((1, 32), i32),
    )(bT)
    counts_g = starts[0, 1:G + 1] - starts[0, :G]
    tg_col = jnp.ceil(RATIO * counts_g.astype(f32)).reshape(G, 1)

    def rank_of(s_row):
        rrow = pl.BlockSpec((1, BR), lambda i: (0, i))
        return pl.pallas_call(
            _rank_body,
            grid=(NRBLK,),
            in_specs=[rrow, rrow, _full((1, NP)), _full((1, NP)), _full((1, 32))],
            out_specs=rrow,
            out_shape=jax.ShapeDtypeStruct((1, NP), i32),
        )(s_row, bT, s_row, bT, starts)

    def conv_fin(acc, hp, inv_row, b, wp):
        return pl.pallas_call(
            _conv_fin_body,
            grid=grid,
            in_specs=[
                pl.BlockSpec((2, BI, D), lambda i: (0, i, 0)),
                _feat(0), _row(0), _full((1, D)), _full((1, D)),
            ],
            out_specs=[_feat(0), _row(0)],
            out_shape=[jax.ShapeDtypeStruct((NP, D), f32),
                       jax.ShapeDtypeStruct((1, NP), f32)],
        )(acc.reshape(2, NP, D), hp, inv_row, b.reshape(1, D), wp.reshape(1, D))

    def sasm(sacc, tp_row, inv_row, b):
        return pl.pallas_call(
            _sasm_body,
            in_specs=[
                pl.BlockSpec((2, NP), lambda: (0, 0)),
                pl.BlockSpec((1, NP), lambda: (0, 0)),
                pl.BlockSpec((1, NP), lambda: (0, 0)),
                pl.BlockSpec((1, 1), lambda: (0, 0)),
            ],
            out_specs=pl.BlockSpec((1, NP), lambda: (0, 0)),
            out_shape=jax.ShapeDtypeStruct((1, NP), f32),
        )(sacc, tp_row, inv_row, b.reshape(1, 1))

    degp = _sc_deg(dst_p)
    hp1, inv_row = pl.pallas_call(
        _m1_body,
        grid=grid,
        in_specs=[_feat(0), _full((D, D)), pl.BlockSpec((2, BI), lambda i: (0, i))],
        out_specs=[_feat(0), _row(0)],
        out_shape=[jax.ShapeDtypeStruct((NP, D), f32),
                   jax.ShapeDtypeStruct((1, NP), f32)],
    )(xp, W1, degp)

    acc1 = _sc_dense(hp1, src_p, dst_p)
    x1, tp1 = conv_fin(acc1, hp1, inv_row, b1, Wp1)
    sacc1 = _sc_scal(tp1.reshape(NP), src_p, dst_p)
    s1 = sasm(sacc1, tp1, inv_row, bp1)
    rank1 = rank_of(s1)

    hp2 = pl.pallas_call(
        _gate_mm_body,
        grid=grid,
        in_specs=[_feat(0), _row(0), _row(0), _row(0), _full((G, 1)), _row(0), _full((D, D))],
        out_specs=_feat(0),
        out_shape=jax.ShapeDtypeStruct((NP, D), f32),
    )(x1, s1, rank1, bT, tg_col, inv_row, W2)
    acc2 = _sc_dense(hp2, src_p, dst_p)
    x2, tp2 = conv_fin(acc2, hp2, inv_row, b2, Wp2)
    sacc2 = _sc_scal(tp2.reshape(NP), src_p, dst_p)
    s2 = sasm(sacc2, tp2, inv_row, bp2)
    rank2 = rank_of(s2)

    tp3 = pl.pallas_call(
        _gate_mv_body,
        grid=grid,
        in_specs=[_feat(0), _row(0), _row(0), _row(0), _full((G, 1)), _row(0), _full((1, D))],
        out_specs=_row(0),
        out_shape=jax.ShapeDtypeStruct((1, NP), f32),
    )(x2, s2, rank2, bT, tg_col, inv_row, W3.reshape(1, D))
    sacc3 = _sc_scal(tp3.reshape(NP), src_p, dst_p)
    x3 = sasm(sacc3, tp3, inv_row, b3)
    rank3 = rank_of(x3)

    def _f0(shape):
        return pl.BlockSpec(shape, lambda: tuple(0 for _ in shape))

    out = pl.pallas_call(
        _final_body,
        in_specs=[
            _f0((1, NP)), _f0((1, NP)), _f0((1, NP)),
            _f0((K, D)), _f0((1, D)), _f0((D, 10)), _f0((1, 10)),
        ],
        out_specs=_f0((G, 10)),
        out_shape=jax.ShapeDtypeStruct((G, 10), f32),
    )(x3, bT, rank3, Wc1, bc1.reshape(1, D), Wc2, bc2.reshape(1, 10))
    return out.reshape(1, -1)


# speedup vs baseline: 1.0417x; 1.0417x over previous
import functools

import jax
import jax.numpy as jnp
from jax import lax
from jax.experimental import pallas as pl
from jax.experimental.pallas import tpu as pltpu
from jax.experimental.pallas import tpu_sc as plsc

N = 10000
E = 320000
D = 128
G = 16
K = 30
RATIO = 0.5

NP = 10240
CH = 128
NW = 32
NCHUNK = 80
EW = NCHUNK * CH
EP = NW * EW
ROWS_W = NP // 16

BI = 512
NBLK = NP // BI
BR = 512
NRBLK = NP // BR
PH = 20

_mesh = plsc.VectorSubcoreMesh(core_axis_name="c", subcore_axis_name="s")


def _zfill16(ref, nwords, value=0.0):
    v = jnp.full((16,), value, jnp.float32)

    def body(i, c):
        ref[pl.ds(i * 16, 16)] = v
        return c

    lax.fori_loop(0, nwords // 16, body, 0)


def _zfill2d(ref):
    z = jnp.zeros((16,), jnp.float32)

    def body(i, c):
        r = i // 8
        q = i % 8
        ref[r, pl.ds(q * 16, 16)] = z
        return c

    lax.fori_loop(0, 128 * 8, body, 0)


@functools.partial(
    pl.kernel,
    out_type=jax.ShapeDtypeStruct((2, NP), jnp.float32),
    mesh=_mesh,
    scratch_types=[
        pltpu.VMEM((NCHUNK, CH), jnp.int32),
        pltpu.VMEM((CH,), jnp.float32),
        pltpu.VMEM((ROWS_W,), jnp.float32),
        pltpu.VMEM_SHARED((NP,), jnp.float32),
        pltpu.SemaphoreType.DMA,
    ],
)
def _sc_deg(dst_hbm, out_hbm, didx, ones, zbuf, acc, sem):
    cid = lax.axis_index("c")
    sid = lax.axis_index("s")
    wid = sid * 2 + cid
    _zfill16(ones, CH, 1.0)
    _zfill16(zbuf, ROWS_W, 0.0)
    pltpu.sync_copy(zbuf, acc.at[pl.ds(sid * ROWS_W, ROWS_W)])
    plsc.subcore_barrier()
    pltpu.sync_copy(dst_hbm.at[pl.ds(wid * NCHUNK, NCHUNK)], didx)

    pend = []
    for ph in range(NCHUNK // PH):
        fired = [pltpu.async_copy(ones, acc.at[didx.at[ph * PH + k]], sem, add=True)
                 for k in range(PH)]
        for cp in pend:
            cp.wait()
        pend = fired
    for cp in pend:
        cp.wait()
    plsc.subcore_barrier()
    pltpu.sync_copy(acc.at[pl.ds(sid * ROWS_W, ROWS_W)],
                    out_hbm.at[cid, pl.ds(sid * ROWS_W, ROWS_W)])


@functools.partial(
    pl.kernel,
    out_type=jax.ShapeDtypeStruct((2, NP), jnp.float32),
    mesh=_mesh,
    scratch_types=[
        pltpu.VMEM((NCHUNK, CH), jnp.int32),
        pltpu.VMEM((NCHUNK, CH), jnp.int32),
        pltpu.VMEM((NCHUNK, CH), jnp.float32),
        pltpu.VMEM((ROWS_W,), jnp.float32),
        pltpu.VMEM_SHARED((NP,), jnp.float32),
        pltpu.SemaphoreType.DMA,
        pltpu.SemaphoreType.DMA,
    ],
)
def _sc_scal(t_hbm, src_hbm, dst_hbm, out_hbm, sidx, didx, vals, zbuf, acc, semg, sems):
    cid = lax.axis_index("c")
    sid = lax.axis_index("s")
    wid = sid * 2 + cid
    _zfill16(zbuf, ROWS_W, 0.0)
    pltpu.sync_copy(zbuf, acc.at[pl.ds(sid * ROWS_W, ROWS_W)])
    plsc.subcore_barrier()
    pltpu.sync_copy(src_hbm.at[pl.ds(wid * NCHUNK, NCHUNK)], sidx)
    pltpu.sync_copy(dst_hbm.at[pl.ds(wid * NCHUNK, NCHUNK)], didx)

    cp_g = {}
    cp_s = {}
    for j in range(PH):
        cp_g[j] = pltpu.async_copy(t_hbm.at[sidx.at[j]], vals.at[j], semg)
    for j in range(NCHUNK):
        cp_g[j].wait()
        cp_s[j] = pltpu.async_copy(vals.at[j], acc.at[didx.at[j]], sems, add=True)
        if j + PH < NCHUNK:
            cp_g[j + PH] = pltpu.async_copy(t_hbm.at[sidx.at[j + PH]], vals.at[j + PH], semg)
    for j in range(NCHUNK):
        cp_s[j].wait()
    plsc.subcore_barrier()
    pltpu.sync_copy(acc.at[pl.ds(sid * ROWS_W, ROWS_W)],
                    out_hbm.at[cid, pl.ds(sid * ROWS_W, ROWS_W)])


NHALF = NCHUNK // 2


@functools.partial(
    pl.kernel,
    out_type=jax.ShapeDtypeStruct((2, NP, D), jnp.float32),
    mesh=_mesh,
    scratch_types=[
        pltpu.VMEM((NHALF, CH), jnp.int32),
        pltpu.VMEM((NHALF, CH), jnp.int32),
        pltpu.VMEM((CH, D), jnp.float32),
        pltpu.VMEM((CH, D), jnp.float32),
        pltpu.VMEM_SHARED((NP, D), jnp.float32),
        pltpu.SemaphoreType.DMA,
        pltpu.SemaphoreType.DMA,
        pltpu.SemaphoreType.DMA,
        pltpu.SemaphoreType.DMA,
    ],
)
def _sc_dense(hp_hbm, src_hbm, dst_hbm, out_hbm, sidx, didx, rows0, rows1,
              acc, semg0, semg1, sems0, sems1):
    cid = lax.axis_index("c")
    sid = lax.axis_index("s")
    wid = sid * 2 + cid
    _zfill2d(rows0)
    for kk in range(ROWS_W // CH):
        pltpu.sync_copy(rows0, acc.at[pl.ds(sid * ROWS_W + kk * CH, CH)])
    plsc.subcore_barrier()

    bufs = (rows0, rows1)
    semg = (semg0, semg1)
    sems = (sems0, sems1)
    for half in range(2):
        base = wid * NCHUNK + half * NHALF
        pltpu.sync_copy(src_hbm.at[pl.ds(base, NHALF)], sidx)
        pltpu.sync_copy(dst_hbm.at[pl.ds(base, NHALF)], didx)
        cp_g = {}
        cp_s = {}
        cp_g[0] = pltpu.async_copy(hp_hbm.at[sidx.at[0]], bufs[0], semg[0])
        for j in range(NHALF):
            p = j % 2
            if j >= 1:
                cp_s[j - 1].wait()
            if j + 1 < NHALF:
                cp_g[j + 1] = pltpu.async_copy(hp_hbm.at[sidx.at[j + 1]], bufs[1 - p], semg[1 - p])
            cp_g[j].wait()
            cp_s[j] = pltpu.async_copy(bufs[p], acc.at[didx.at[j]], sems[p], add=True)
        cp_s[NHALF - 1].wait()
    plsc.subcore_barrier()
    pltpu.sync_copy(acc.at[pl.ds(sid * ROWS_W, ROWS_W)],
                    out_hbm.at[cid, pl.ds(sid * ROWS_W, ROWS_W)])


def _elu(a):
    return jnp.where(a > 0, a, jnp.exp(a) - 1.0)


def _to_col(row):
    B = row.shape[1]
    eye = (lax.broadcasted_iota(jnp.int32, (B, B), 0)
           == lax.broadcasted_iota(jnp.int32, (B, B), 1))
    z = jnp.zeros((B, B), row.dtype)
    return jnp.sum(jnp.where(eye, jnp.broadcast_to(row, (B, B)), z),
                   axis=1, keepdims=True)


def _to_row(col):
    B = col.shape[0]
    eye = (lax.broadcasted_iota(jnp.int32, (B, B), 0)
           == lax.broadcasted_iota(jnp.int32, (B, B), 1))
    z = jnp.zeros((B, B), col.dtype)
    return jnp.sum(jnp.where(eye, jnp.broadcast_to(col, (B, B)), z),
                   axis=0, keepdims=True)


def _starts_body(bT_ref, st_ref):
    def body(t, acc):
        blk = bT_ref[:, pl.ds(t * BI, BI)]
        gi = lax.broadcasted_iota(jnp.int32, (32, BI), 0)
        lt = (jnp.broadcast_to(blk, (32, BI)) < gi).astype(jnp.int32)
        return acc + jnp.sum(lt, axis=1, keepdims=True)

    st = lax.fori_loop(0, NBLK, body, jnp.zeros((32, 1), jnp.int32))
    st_ref[...] = _to_row(st)


def _m1_body(x_ref, w_ref, degp_ref, hp_ref, inv_ref):
    deg = degp_ref[0:1, :] + degp_ref[1:2, :] + 1.0
    inv_row = lax.rsqrt(deg)
    inv_ref[...] = inv_row
    h = jnp.dot(x_ref[...], w_ref[...], preferred_element_type=jnp.float32)
    hp_ref[...] = h * _to_col(inv_row)


def _conv_fin_body(acc_ref, hp_ref, inv_ref, b_ref, wp_ref, x_ref, tp_ref):
    inv_col = _to_col(inv_ref[...])
    a = inv_col * (acc_ref[0] + acc_ref[1] + hp_ref[...]) + b_ref[...]
    x = _elu(a)
    x_ref[...] = x
    t = jnp.sum(x * wp_ref[...], axis=1, keepdims=True) * inv_col
    tp_ref[...] = _to_row(t)


def _rank_body(sacc_ref, tp_ref, inv_ref, bp_ref, b_ref, bT_ref, st_ref, s_ref, rank_ref, s_scr):
    i = pl.program_id(0)
    s_full = (inv_ref[...] * (sacc_ref[0:1, :] + sacc_ref[1:2, :] + tp_ref[...])
              + bp_ref[...])
    s_scr[...] = s_full
    i0 = pl.multiple_of(i * BR, BR)
    s_blk = s_scr[:, pl.ds(i0, BR)]
    s_ref[...] = s_blk
    si = _to_col(s_blk)
    bi = _to_col(b_ref[...])
    glo = jnp.min(b_ref[...])
    ghi = jnp.max(b_ref[...])
    gidx = lax.broadcasted_iota(jnp.int32, (1, 32), 1)
    srow = st_ref[...]
    z = jnp.zeros((1, 32), jnp.int32)
    jlo_val = jnp.sum(jnp.where(gidx == glo, srow, z))
    jhi_val = jnp.sum(jnp.where(gidx == ghi + 1, srow, z))
    jlo = jlo_val // BR
    njb = (jhi_val + BR - 1) // BR - jlo
    ii = lax.broadcasted_iota(jnp.int32, (BR, 1), 0) + i * BR

    def body(kk, cnt):
        j0 = pl.multiple_of((jlo + kk) * BR, BR)
        sj = s_scr[:, pl.ds(j0, BR)]
        bj = bT_ref[:, pl.ds(j0, BR)]
        jj = lax.broadcasted_iota(jnp.int32, (1, BR), 1) + j0
        ahead = (sj > si) | ((sj == si) & (jj < ii))
        m = (bj == bi) & ahead
        return cnt + jnp.sum(m.astype(jnp.float32), axis=1, keepdims=True)

    cnt = lax.fori_loop(0, njb, body, jnp.zeros((BR, 1), jnp.float32))
    rank_ref[...] = _to_row(cnt.astype(jnp.int32))


def _gate(s_row, rank_row, b_row, tg_ref):
    geq = (lax.broadcasted_iota(jnp.int32, (G, BI), 0)
           == jnp.broadcast_to(b_row, (G, BI)))
    zf = jnp.zeros((G, BI), jnp.float32)
    thr = jnp.sum(jnp.where(geq, jnp.broadcast_to(tg_ref[...], (G, BI)), zf),
                  axis=0, keepdims=True)
    keep = (rank_row.astype(jnp.float32) < thr).astype(jnp.float32)
    return _to_col(jnp.tanh(s_row) * keep)


def _gate_mm_body(x_ref, s_ref, rank_ref, b_ref, tg_ref, inv_ref, w_ref, out_ref):
    gcol = _gate(s_ref[...], rank_ref[...], b_ref[...], tg_ref)
    xp = x_ref[...] * gcol
    out_ref[...] = (jnp.dot(xp, w_ref[...], preferred_element_type=jnp.float32)
                    * _to_col(inv_ref[...]))


def _gate_mv_body(x_ref, s_ref, rank_ref, b_ref, tg_ref, inv_ref, w_ref, tp_ref):
    gcol = _gate(s_ref[...], rank_ref[...], b_ref[...], tg_ref)
    xp = x_ref[...] * gcol
    t = jnp.sum(xp * w_ref[...], axis=1, keepdims=True) * _to_col(inv_ref[...])
    tp_ref[...] = _to_row(t)


def _final_body(x3T_ref, bT_ref, r3_ref, wc1_ref, bc1_ref, wc2_ref, bc2_ref, out_ref):
    def body(jb, acc):
        j0 = pl.multiple_of(jb * BI, BI)
        xr = x3T_ref[:, pl.ds(j0, BI)]
        bt = bT_ref[:, pl.ds(j0, BI)]
        rk = _to_col(r3_ref[:, pl.ds(j0, BI)])
        gi = lax.broadcasted_iota(jnp.int32, (G, BI), 0)
        A = jnp.where(bt == gi, jnp.broadcast_to(xr, (G, BI)), 0.0)
        ki = lax.broadcasted_iota(jnp.int32, (BI, K), 1)
        B = (rk == ki).astype(jnp.float32)
        return acc + jnp.dot(A, B, preferred_element_type=jnp.float32)

    pooled = lax.fori_loop(0, NBLK, body, jnp.zeros((G, K), jnp.float32))
    h = jnp.dot(pooled, wc1_ref[...], preferred_element_type=jnp.float32) + bc1_ref[...]
    h = _elu(h)
    out_ref[...] = jnp.dot(h, wc2_ref[...], preferred_element_type=jnp.float32) + bc2_ref[...]


def _row(i):
    return pl.BlockSpec((1, BI), lambda i: (0, i))


def _feat(i):
    return pl.BlockSpec((BI, D), lambda i: (i, 0))


def _full(shape):
    return pl.BlockSpec(shape, lambda i: tuple(0 for _ in shape))


def kernel(x, edge_index, batch, W1, b1, Wp1, bp1, W2, b2, Wp2, bp2, W3, b3, Wc1, bc1, Wc2, bc2):
    f32 = jnp.float32
    i32 = jnp.int32
    src = edge_index[0].astype(i32)
    dst = edge_index[1].astype(i32)
    pe = EP - E
    pad_ar = jnp.arange(pe, dtype=i32)
    src_p = jnp.concatenate([src, pad_ar % N]).reshape(EP // CH, CH)
    dst_p = jnp.concatenate([dst, N + pad_ar % (NP - N)]).reshape(EP // CH, CH)

    xp = jnp.pad(x, ((0, NP - N), (0, 0)))
    batch_p = jnp.concatenate([batch.astype(i32), jnp.full((NP - N,), G, i32)])
    bT = batch_p.reshape(1, NP)

    grid = (NBLK,)

    starts = pl.pallas_call(
        _starts_body,
        in_specs=[pl.BlockSpec((1, NP), lambda: (0, 0))],
        out_specs=pl.BlockSpec((1, 32), lambda: (0, 0)),
        out_shape=jax.ShapeDtypeStruct((1, 32), i32),
    )(bT)
    counts_g = starts[0, 1:G + 1] - starts[0, :G]
    tg_col = jnp.ceil(RATIO * counts_g.astype(f32)).reshape(G, 1)

    def rank_of(sacc, tp_row, inv_row, bp):
        rrow = pl.BlockSpec((1, BR), lambda i: (0, i))
        return pl.pallas_call(
            _rank_body,
            grid=(NRBLK,),
            in_specs=[_full((2, NP)), _full((1, NP)), _full((1, NP)), _full((1, 1)),
                      rrow, _full((1, NP)), _full((1, 32))],
            out_specs=[rrow, rrow],
            out_shape=[jax.ShapeDtypeStruct((1, NP), f32),
                       jax.ShapeDtypeStruct((1, NP), i32)],
            scratch_shapes=[pltpu.VMEM((1, NP), f32)],
        )(sacc, tp_row, inv_row, bp.reshape(1, 1), bT, bT, starts)

    def conv_fin(acc, hp, inv_row, b, wp):
        return pl.pallas_call(
            _conv_fin_body,
            grid=grid,
            in_specs=[
                pl.BlockSpec((2, BI, D), lambda i: (0, i, 0)),
                _feat(0), _row(0), _full((1, D)), _full((1, D)),
            ],
            out_specs=[_feat(0), _row(0)],
            out_shape=[jax.ShapeDtypeStruct((NP, D), f32),
                       jax.ShapeDtypeStruct((1, NP), f32)],
        )(acc.reshape(2, NP, D), hp, inv_row, b.reshape(1, D), wp.reshape(1, D))

    degp = _sc_deg(dst_p)
    hp1, inv_row = pl.pallas_call(
        _m1_body,
        grid=grid,
        in_specs=[_feat(0), _full((D, D)), pl.BlockSpec((2, BI), lambda i: (0, i))],
        out_specs=[_feat(0), _row(0)],
        out_shape=[jax.ShapeDtypeStruct((NP, D), f32),
                   jax.ShapeDtypeStruct((1, NP), f32)],
    )(xp, W1, degp)

    acc1 = _sc_dense(hp1, src_p, dst_p)
    x1, tp1 = conv_fin(acc1, hp1, inv_row, b1, Wp1)
    sacc1 = _sc_scal(tp1.reshape(NP), src_p, dst_p)
    s1, rank1 = rank_of(sacc1, tp1, inv_row, bp1)

    hp2 = pl.pallas_call(
        _gate_mm_body,
        grid=grid,
        in_specs=[_feat(0), _row(0), _row(0), _row(0), _full((G, 1)), _row(0), _full((D, D))],
        out_specs=_feat(0),
        out_shape=jax.ShapeDtypeStruct((NP, D), f32),
    )(x1, s1, rank1, bT, tg_col, inv_row, W2)
    acc2 = _sc_dense(hp2, src_p, dst_p)
    x2, tp2 = conv_fin(acc2, hp2, inv_row, b2, Wp2)
    sacc2 = _sc_scal(tp2.reshape(NP), src_p, dst_p)
    s2, rank2 = rank_of(sacc2, tp2, inv_row, bp2)

    tp3 = pl.pallas_call(
        _gate_mv_body,
        grid=grid,
        in_specs=[_feat(0), _row(0), _row(0), _row(0), _full((G, 1)), _row(0), _full((1, D))],
        out_specs=_row(0),
        out_shape=jax.ShapeDtypeStruct((1, NP), f32),
    )(x2, s2, rank2, bT, tg_col, inv_row, W3.reshape(1, D))
    sacc3 = _sc_scal(tp3.reshape(NP), src_p, dst_p)
    x3, rank3 = rank_of(sacc3, tp3, inv_row, b3)

    def _f0(shape):
        return pl.BlockSpec(shape, lambda: tuple(0 for _ in shape))

    out = pl.pallas_call(
        _final_body,
        in_specs=[
            _f0((1, NP)), _f0((1, NP)), _f0((1, NP)),
            _f0((K, D)), _f0((1, D)), _f0((D, 10)), _f0((1, 10)),
        ],
        out_specs=_f0((G, 10)),
        out_shape=jax.ShapeDtypeStruct((G, 10), f32),
    )(x3, bT, rank3, Wc1, bc1.reshape(1, D), Wc2, bc2.reshape(1, 10))
    return out.reshape(1, -1)
